# baseline (speedup 1.0000x reference)
import functools

import jax
import jax.numpy as jnp
from jax import lax
from jax.experimental import pallas as pl
from jax.experimental.pallas import tpu as pltpu
from jax.experimental.pallas import tpu_sc as plsc

B = 16384
D = 128
C = 100000
ALPHA = 0.1

NC = 2
NS = 16
NW = NC * NS
HALF = B // NC
PT = B // NW
CH = 128
NCH = PT // CH
NV = D // 16

_mesh = plsc.VectorSubcoreMesh(
    core_axis_name="c", subcore_axis_name="s", num_cores=NC, num_subcores=NS)


def _zero_rows(buf, rows, width):
    def body(r):
        for v in range(width // 16):
            buf[r, pl.ds(v * 16, 16)] = jnp.zeros((16,), buf.dtype)
    pl.loop(0, rows)(body)


@functools.partial(
    pl.kernel,
    out_type=(
        jax.ShapeDtypeStruct((NC * C, 16), jnp.int32),
        jax.ShapeDtypeStruct((B, D), jnp.float32),
        jax.ShapeDtypeStruct((B, 16), jnp.float32),
    ),
    mesh=_mesh,
    compiler_params=pltpu.CompilerParams(needs_layout_passes=False, use_tc_tiling_on_sc=False),
    scratch_types=(
        pltpu.VMEM_SHARED((HALF, D), jnp.float32),
        pltpu.VMEM_SHARED((HALF, 16), jnp.float32),
        pltpu.VMEM((CH, D), jnp.float32),
        pltpu.VMEM((CH, 16), jnp.float32),
        pltpu.VMEM((NCH, CH), jnp.int32),
        pltpu.VMEM((NCH, CH), jnp.int32),
        pltpu.VMEM((NCH, CH), jnp.int32),
        pltpu.VMEM((CH, 16), jnp.int32),
        pltpu.VMEM((CH, 16), jnp.int32),
    ),
)
def _l1(features, labels2, rep_out, sums_out, meta_out,
        sums_sp, meta_sp, feat_v, meta_v, lab_v, laboff_v, slot_v, irows, srows):
    c = lax.axis_index("c")
    s = lax.axis_index("s")
    half_base = c * HALF
    my = half_base + s * PT
    row0 = (c * NS + s) * NCH

    _zero_rows(feat_v, CH, D)
    _zero_rows(meta_v, CH, 16)
    for k in range(NCH):
        pltpu.sync_copy(feat_v, sums_sp.at[pl.ds(s * PT + k * CH, CH)])
        pltpu.sync_copy(meta_v, meta_sp.at[pl.ds(s * PT + k * CH, CH)])

    iota16 = lax.iota(jnp.int32, 16)
    zeros16 = jnp.zeros((16,), jnp.int32)

    for j in range(CH // 16):
        plsc.store_scatter(meta_v, [iota16 + j * 16, zeros16],
                           jnp.ones((16,), jnp.float32))

    pltpu.sync_copy(labels2.at[pl.ds(row0, NCH)], lab_v)
    for k in range(NCH):
        for v in range(CH // 16):
            sl = pl.ds(v * 16, 16)
            laboff_v[k, sl] = lab_v[k, sl] + c * C

    for k in range(NCH):
        def put_idx(r, _k=k):
            irows[r, pl.ds(0, 16)] = (
                jnp.zeros((16,), jnp.int32) + (my + _k * CH + r))
        pl.loop(0, CH)(put_idx)
        pltpu.sync_copy(irows, rep_out.at[laboff_v.at[k]])

    plsc.subcore_barrier()

    for k in range(NCH):
        pltpu.sync_copy(rep_out.at[laboff_v.at[k]], srows)
        for j in range(CH // 16):
            rid = iota16 + j * 16
            col0 = plsc.load_gather(srows, [rid, zeros16])
            slot_v[k, pl.ds(j * 16, 16)] = col0 - half_base

    for k in range(NCH):
        pltpu.sync_copy(features.at[pl.ds(my + k * CH, CH)], feat_v)
        pltpu.sync_copy(feat_v, sums_sp.at[slot_v.at[k]], add=True)

        for j in range(CH // 16):
            sl = pl.ds(j * 16, 16)
            rid = iota16 + j * 16
            gidx = slot_v[k, sl] + half_base
            own = iota16 + (my + k * CH + j * 16)
            labf = lab_v[k, sl].astype(jnp.float32)
            val = jnp.where(gidx == own, labf, jnp.float32(0.0))
            plsc.store_scatter(meta_v, [rid, zeros16 + 1], val)
        pltpu.sync_copy(meta_v, meta_sp.at[slot_v.at[k]], add=True)

    plsc.subcore_barrier()

    for k in range(NCH):
        rows = pl.ds(s * PT + k * CH, CH)
        out_rows = pl.ds(half_base + s * PT + k * CH, CH)
        pltpu.sync_copy(sums_sp.at[rows], feat_v)
        pltpu.sync_copy(feat_v, sums_out.at[out_rows])
        pltpu.sync_copy(meta_sp.at[rows], meta_v)
        pltpu.sync_copy(meta_v, meta_out.at[out_rows])


@functools.partial(
    pl.kernel,
    out_type=jax.ShapeDtypeStruct((NW, CH), jnp.float32),
    mesh=_mesh,
    compiler_params=pltpu.CompilerParams(needs_layout_passes=False, use_tc_tiling_on_sc=False),
    scratch_types=(
        pltpu.VMEM((NCH, CH), jnp.int32),
        pltpu.VMEM((NCH, CH), jnp.int32),
        pltpu.VMEM((1, CH), jnp.int32),
        pltpu.VMEM((1, CH), jnp.int32),
        pltpu.VMEM((CH, 16), jnp.int32),
        pltpu.VMEM((CH, 16), jnp.int32),
        pltpu.VMEM((CH, D), jnp.float32),
        pltpu.VMEM((CH, D), jnp.float32),
        pltpu.VMEM((CH, 16), jnp.float32),
        pltpu.VMEM((CH, 16), jnp.float32),
        pltpu.VMEM((CH, D), jnp.float32),
        pltpu.VMEM((CH, D), jnp.float32),
        pltpu.VMEM((CH, D), jnp.float32),
        pltpu.VMEM((1, CH), jnp.float32),
    ),
)
def _l2(features, labels2, centers, rep_in, sums_in, meta_in, cpy,
        partials,
        lab_v, labB_v, idxA, idxB, srowsA, srowsB, rowsA, rowsB,
        metaA, metaB, crows, frows, obuf, pacc):
    c = lax.axis_index("c")
    s = lax.axis_index("s")
    wid = s * NC + c
    my = wid * PT
    row0 = wid * NCH
    iota16 = lax.iota(jnp.int32, 16)
    zeros16 = jnp.zeros((16,), jnp.int32)

    pltpu.sync_copy(labels2.at[pl.ds(row0, NCH)], lab_v)
    for k in range(NCH):
        for v in range(CH // 16):
            sl = pl.ds(v * 16, 16)
            labB_v[k, sl] = lab_v[k, sl] + C

    def chunk_body(k, acc):
        pltpu.sync_copy(rep_in.at[lab_v.at[k]], srowsA)
        pltpu.sync_copy(rep_in.at[labB_v.at[k]], srowsB)

        for j in range(CH // 16):
            sl = pl.ds(j * 16, 16)
            rid = iota16 + j * 16
            ga = plsc.load_gather(srowsA, [rid, zeros16])
            gb = plsc.load_gather(srowsB, [rid, zeros16])
            idxA[0, sl] = jnp.minimum(jnp.maximum(ga, 0), HALF - 1)
            idxB[0, sl] = jnp.minimum(jnp.maximum(gb, HALF), B - 1)

        pltpu.sync_copy(sums_in.at[idxA.at[0]], rowsA)
        pltpu.sync_copy(meta_in.at[idxA.at[0]], metaA)
        pltpu.sync_copy(sums_in.at[idxB.at[0]], rowsB)
        pltpu.sync_copy(meta_in.at[idxB.at[0]], metaB)
        pltpu.sync_copy(centers.at[lab_v.at[k]], crows)
        pltpu.sync_copy(features.at[pl.ds(my + k * CH, CH)], frows)

        def group_body(j, acc2, _k=k):
            rid = iota16 + j * 16
            labf = plsc.load_gather(lab_v, [zeros16 + _k, rid]).astype(jnp.float32)
            laA = plsc.load_gather(metaA, [rid, zeros16 + 1])
            cntA = plsc.load_gather(metaA, [rid, zeros16])
            laB = plsc.load_gather(metaB, [rid, zeros16 + 1])
            cntB = plsc.load_gather(metaB, [rid, zeros16])
            one = jnp.float32(1.0)
            zero = jnp.float32(0.0)
            wA = jnp.where(laA == labf, one, zero)
            wB = jnp.where(laB == labf, one, zero)
            cnt = wA * cntA + wB * cntB
            scv = jnp.float32(ALPHA) / cnt
            scA = scv * wA
            scB = scv * wB
            for i in range(16):
                r = j * 16 + i
                wa = scA[i]
                wb = scB[i]
                for v in range(NV):
                    sl = pl.ds(v * 16, 16)
                    cv = crows[r, sl]
                    fv = frows[r, sl]
                    obuf[r, sl] = (cv * jnp.float32(1.0 - ALPHA)
                                   + wa * rowsA[r, sl] + wb * rowsB[r, sl])
                    dv = fv - cv
                    acc2 = acc2 + dv * dv
            return acc2
        acc = pl.loop(0, CH // 16, init_carry=acc)(group_body)

        pltpu.sync_copy(obuf, cpy.at[lab_v.at[k]])
        return acc

    acc = pl.loop(0, NCH, init_carry=jnp.zeros((16,), jnp.float32))(chunk_body)

    for v in range(CH // 16):
        pacc[0, pl.ds(v * 16, 16)] = jnp.zeros((16,), jnp.float32)
    pacc[0, pl.ds(0, 16)] = acc
    pltpu.sync_copy(pacc, partials.at[pl.ds(wid, 1)])


def _l3_body(p_ref, o_ref):
    o_ref[...] = jnp.broadcast_to(jnp.sum(p_ref[...]) * (1.0 / B), (8, 128))


def kernel(features, labels, centers):
    labels = labels.astype(jnp.int32)
    labels2 = labels.reshape(B // CH, CH)

    rep, sums, meta = _l1(features, labels2)

    cpy = jax.new_ref(centers)
    partials = _l2(features, labels2, centers, rep, sums, meta, cpy)
    new_centers = cpy[...]

    lossmat = pl.pallas_call(
        _l3_body,
        out_shape=jax.ShapeDtypeStruct((8, 128), jnp.float32),
    )(partials)
    return lossmat[0, 0], new_centers
